# baseline (speedup 1.0000x reference)
import functools
import jax
import jax.numpy as jnp
from jax import lax
from jax.experimental import pallas as pl
from jax.experimental.pallas import tpu as pltpu
from jax.experimental.pallas import tpu_sc as plsc

L = 16
NC = 2
NS = 16
NW = NC * NS
B = 256
R = 5120
SHARE = R // NS
ZR = 160
D = 128


def _sc_scatter_sums(table, heads, tails, etypes, wtable, n_out):
    E = heads.shape[0]
    C = -(-E // (NS * B))
    EW = C * B
    E_pad = EW * NS
    NPASS = -(-n_out // (2 * R))
    n_pad = NPASS * 2 * R

    pad = E_pad - E
    heads = jnp.concatenate([heads, jnp.full((pad,), -1, jnp.int32)])
    tails = jnp.concatenate([tails, jnp.zeros((pad,), jnp.int32)])
    etypes = jnp.concatenate([etypes, jnp.zeros((pad,), jnp.int32)])

    mesh = plsc.VectorSubcoreMesh(core_axis_name="c", subcore_axis_name="s")

    @functools.partial(
        pl.kernel,
        mesh=mesh,
        out_type=[
            jax.ShapeDtypeStruct((n_pad, D), jnp.float32),
            jax.ShapeDtypeStruct((n_pad, D), jnp.float32),
        ],
        scratch_types=[
            pltpu.VMEM((B,), jnp.int32),
            pltpu.VMEM((B,), jnp.int32),
            pltpu.VMEM((B,), jnp.int32),
            pltpu.VMEM((B,), jnp.int32),
            pltpu.VMEM((B, D), jnp.float32),
            pltpu.VMEM((B, D), jnp.float32),
            pltpu.VMEM((ZR, D), jnp.float32),
            pltpu.VMEM_SHARED((R + 16, D), jnp.float32),
            pltpu.SemaphoreType.DMA,
        ],
    )
    def k(heads_h, tails_h, ets_h, table_h, wt_h, sums_o, cnt_o,
          hv, tv, ev, iv, rows, wrows, zv, ssh, sem):
        c = lax.axis_index("c")
        s = lax.axis_index("s")

        @pl.loop(0, ZR)
        def _(r):
            for j in range(D // L):
                zv[r, pl.ds(j * L, L)] = jnp.zeros((L,), jnp.float32)

        def zero_share():
            for q in range(SHARE // ZR):
                pltpu.sync_copy(zv, ssh.at[pl.ds(s * SHARE + q * ZR, ZR)])

        def local_idx(base):
            for t in range(B // L):
                sl = pl.ds(t * L, L)
                lidx = hv[sl] - base
                ok = (lidx >= 0) & (lidx < R)
                iv[sl] = jnp.where(ok, lidx, R)

        def writeback(p, out_h):
            plsc.subcore_barrier()
            g0 = p * (2 * R) + c * R + s * SHARE
            pltpu.sync_copy(ssh.at[pl.ds(s * SHARE, SHARE)],
                            out_h.at[pl.ds(g0, SHARE)])
            plsc.subcore_barrier()
            zero_share()
            plsc.subcore_barrier()

        zero_share()
        plsc.subcore_barrier()

        @pl.loop(0, NPASS)
        def _(p):
            base = p * (2 * R) + c * R

            @pl.loop(0, C)
            def _(kk):
                off = s * EW + kk * B
                pltpu.sync_copy(heads_h.at[pl.ds(off, B)], hv)
                pltpu.sync_copy(tails_h.at[pl.ds(off, B)], tv)
                pltpu.sync_copy(ets_h.at[pl.ds(off, B)], ev)
                pltpu.async_copy(table_h.at[tv], rows, sem).wait()
                pltpu.async_copy(wt_h.at[ev], wrows, sem).wait()

                @pl.loop(0, B)
                def _(r):
                    for j in range(D // L):
                        sl = pl.ds(j * L, L)
                        rows[r, sl] = rows[r, sl] * wrows[r, sl]

                local_idx(base)
                pltpu.sync_copy(rows, ssh.at[iv], add=True)

            writeback(p, sums_o)

        @pl.loop(0, B)
        def _(r):
            for j in range(D // L):
                rows[r, pl.ds(j * L, L)] = jnp.ones((L,), jnp.float32)

        @pl.loop(0, NPASS)
        def _(p):
            base = p * (2 * R) + c * R

            @pl.loop(0, C)
            def _(kk):
                off = s * EW + kk * B
                pltpu.sync_copy(heads_h.at[pl.ds(off, B)], hv)
                local_idx(base)
                pltpu.sync_copy(rows, ssh.at[iv], add=True)

            writeback(p, cnt_o)

    return k(heads, tails, etypes, table, wtable)


def _divide_kernel(sums_ref, cnt_ref, out_ref):
    cnt = jnp.maximum(cnt_ref[:, :1], 1.0)
    out_ref[:, :] = sums_ref[:, :] / cnt


def _scatter_mean(table, heads, tails, etypes, wtable, n_out):
    sums, cnt = _sc_scatter_sums(table, heads, tails, etypes, wtable, n_out)
    n_pad = sums.shape[0]
    BLK = SHARE
    out = pl.pallas_call(
        _divide_kernel,
        grid=(n_pad // BLK,),
        in_specs=[
            pl.BlockSpec((BLK, D), lambda i: (i, 0)),
            pl.BlockSpec((BLK, D), lambda i: (i, 0)),
        ],
        out_specs=pl.BlockSpec((BLK, D), lambda i: (i, 0)),
        out_shape=jax.ShapeDtypeStruct((n_pad, D), jnp.float32),
    )(sums, cnt)
    return out[:n_out]


def kernel(entity_emb, user_emb, edge_index, edge_type,
           extra_edge_index, extra_edge_type, weight, extra_weight):
    n_ent = entity_emb.shape[0]
    n_rel = weight.shape[0]
    kg_type = (edge_type - 1) % n_rel
    entity_agg = _scatter_mean(entity_emb, edge_index[0], edge_index[1],
                               kg_type, weight, n_ent)

    all_embed = jnp.concatenate([user_emb, entity_emb], axis=0)
    n_nodes = all_embed.shape[0]
    node_agg = _scatter_mean(all_embed, extra_edge_index[0],
                             extra_edge_index[1], extra_edge_type,
                             extra_weight, n_nodes)
    return (entity_agg, node_agg)
